# baseline (speedup 1.0000x reference)
import functools

import jax
import jax.numpy as jnp
from jax import lax
from jax.experimental import pallas as pl
from jax.experimental.pallas import tpu as pltpu
from jax.experimental.pallas import tpu_sc as plsc

_CHUNK = 128


def _make_gather(B, D, NC, NS):
    NW = NC * NS
    b_per_w = B // NW
    n_chunks = b_per_w // _CHUNK
    mesh = plsc.VectorSubcoreMesh(core_axis_name="c", subcore_axis_name="s")

    @functools.partial(
        pl.kernel,
        mesh=mesh,
        compiler_params=pltpu.CompilerParams(use_tc_tiling_on_sc=False),
        out_type=[
            jax.ShapeDtypeStruct((B, D), jnp.float32),
            jax.ShapeDtypeStruct((B, D), jnp.float32),
        ],
        scratch_types=[
            pltpu.VMEM((b_per_w,), jnp.int32),
            pltpu.VMEM((b_per_w, D), jnp.float32),
            pltpu.VMEM((b_per_w, D), jnp.float32),
            pltpu.SemaphoreType.DMA,
            pltpu.SemaphoreType.DMA,
        ],
    )
    def gather2(idx_hbm, ktab_hbm, vtab_hbm, kout_hbm, vout_hbm,
                idx_v, krows_v, vrows_v, ksem, vsem):
        wid = lax.axis_index("s") * NC + lax.axis_index("c")
        base = wid * b_per_w
        pltpu.sync_copy(idx_hbm.at[pl.ds(base, b_per_w)], idx_v)
        kcopies = []
        vcopies = []
        for j in range(n_chunks):
            sl = pl.ds(j * _CHUNK, _CHUNK)
            kcopies.append(
                pltpu.async_copy(ktab_hbm.at[idx_v.at[sl]], krows_v.at[sl], ksem))
            vcopies.append(
                pltpu.async_copy(vtab_hbm.at[idx_v.at[sl]], vrows_v.at[sl], vsem))
        for c in kcopies:
            c.wait()
        pltpu.sync_copy(krows_v, kout_hbm.at[pl.ds(base, b_per_w)])
        for c in vcopies:
            c.wait()
        pltpu.sync_copy(vrows_v, vout_hbm.at[pl.ds(base, b_per_w)])

    return gather2


def kernel(x, key_table, value_table):
    B = x.shape[0]
    D = key_table.shape[1]
    info = plsc.get_sparse_core_info()
    fn = _make_gather(B, D, info.num_cores, info.num_subcores)
    xi = x.astype(jnp.int32).reshape(-1)
    kout, vout = fn(xi, key_table, value_table)
    return (kout, vout)
